# baseline (speedup 1.0000x reference)
import functools

import jax
import jax.numpy as jnp
from jax import lax
from jax.experimental import pallas as pl
from jax.experimental.pallas import tpu as pltpu
from jax.experimental.pallas import tpu_sc as plsc

_BATCH = 16384

_info = plsc.get_sparse_core_info()
_NC = _info.num_cores
_NS = _info.num_subcores
_NW = _NC * _NS
_B_PER_W = _BATCH // _NW
_CHUNK = 128
_NCHUNK = _B_PER_W // _CHUNK
_LANES = 16


_FLAT_BLK = 393216


_TH_BLK = 131072


def _flatten_pair_body(th_ref, be_ref, oth_ref, obe_ref):
    i = pl.program_id(0)

    @pl.when(i == 0)
    def _():
        oth_ref[...] = th_ref[...].reshape(oth_ref.shape)

    obe_ref[...] = be_ref[...].reshape(obe_ref.shape)


def _flatten_pair(theta_t, beta_t):
    nt = theta_t.shape[1]
    nb = beta_t.shape[1]
    return pl.pallas_call(
        _flatten_pair_body,
        grid=(pl.cdiv(nb, _FLAT_BLK),),
        in_specs=[
            pl.BlockSpec((1, _TH_BLK), lambda i: (0, 0)),
            pl.BlockSpec((1, _FLAT_BLK), lambda i: (0, i)),
        ],
        out_specs=[
            pl.BlockSpec((_TH_BLK,), lambda i: (0,)),
            pl.BlockSpec((_FLAT_BLK,), lambda i: (i,)),
        ],
        out_shape=[
            jax.ShapeDtypeStruct((nt,), jnp.float32),
            jax.ShapeDtypeStruct((nb,), jnp.float32),
        ],
    )(theta_t, beta_t)


@functools.partial(
    pl.kernel,
    mesh=plsc.VectorSubcoreMesh(core_axis_name="c", subcore_axis_name="s"),
    out_type=jax.ShapeDtypeStruct((_BATCH,), jnp.float32),
    scratch_types=[
        pltpu.VMEM((_B_PER_W,), jnp.int32),
        pltpu.VMEM((_B_PER_W,), jnp.int32),
        pltpu.VMEM((_B_PER_W,), jnp.float32),
        pltpu.VMEM((_B_PER_W,), jnp.float32),
        pltpu.SemaphoreType.DMA,
        pltpu.SemaphoreType.DMA,
        [pltpu.SemaphoreType.DMA] * _NCHUNK,
        [pltpu.SemaphoreType.DMA] * _NCHUNK,
        pltpu.SemaphoreType.DMA,
    ],
)
def _irt_sc(aidx_hbm, tidx_hbm, th_hbm, be_hbm, out_hbm,
            aidx_v, tidx_v, th_v, be_v, sem_ai, sem_ti, sems_th, sems_be,
            sem_o):
    wid = lax.axis_index("s") * _NC + lax.axis_index("c")
    base = wid * _B_PER_W

    cp_a = pltpu.async_copy(aidx_hbm.at[pl.ds(base, _B_PER_W)], aidx_v, sem_ai)
    cp_t = pltpu.async_copy(tidx_hbm.at[pl.ds(base, _B_PER_W)], tidx_v, sem_ti)

    cp_a.wait()
    th_cps = []
    for j in range(_NCHUNK):
        sl = pl.ds(j * _CHUNK, _CHUNK)
        th_cps.append(
            pltpu.async_copy(th_hbm.at[aidx_v.at[sl]], th_v.at[sl], sems_th[j]))
    cp_t.wait()
    be_cps = []
    for j in range(_NCHUNK):
        sl = pl.ds(j * _CHUNK, _CHUNK)
        be_cps.append(
            pltpu.async_copy(be_hbm.at[tidx_v.at[sl]], be_v.at[sl], sems_be[j]))

    out_cps = []
    for j in range(_NCHUNK):
        th_cps[j].wait()
        be_cps[j].wait()
        for i in range(_CHUNK // _LANES):
            sl = pl.ds(j * _CHUNK + i * _LANES, _LANES)
            th_v[sl] = th_v[sl] - be_v[sl]
        sl = pl.ds(j * _CHUNK, _CHUNK)
        out_cps.append(
            pltpu.async_copy(
                th_v.at[sl], out_hbm.at[pl.ds(base + j * _CHUNK, _CHUNK)],
                sem_o))
    for cp in out_cps:
        cp.wait()


def kernel(agent_idx, task_idx, theta, beta):
    agent_idx = agent_idx.astype(jnp.int32)
    task_idx = task_idx.astype(jnp.int32)
    th_flat, be_flat = _flatten_pair(theta.T, beta.T)
    return _irt_sc(agent_idx, task_idx, th_flat, be_flat)


# speedup vs baseline: 1.0617x; 1.0617x over previous
import functools

import jax
import jax.numpy as jnp
from jax import lax
from jax.experimental import pallas as pl
from jax.experimental.pallas import tpu as pltpu
from jax.experimental.pallas import tpu_sc as plsc

_BATCH = 16384

_info = plsc.get_sparse_core_info()
_NC = _info.num_cores
_NS = _info.num_subcores
_NW = _NC * _NS
_B_PER_W = _BATCH // _NW
_CHUNK = 128
_NCHUNK = _B_PER_W // _CHUNK
_LANES = 16


_FLAT_BLK = 507904


_TH_BLK = 131072


def _flatten_pair_body(th_ref, be_ref, oth_ref, obe_ref):
    i = pl.program_id(0)

    @pl.when(i == 0)
    def _():
        oth_ref[...] = th_ref[...].reshape(oth_ref.shape)

    obe_ref[...] = be_ref[...].reshape(obe_ref.shape)


def _flatten_pair(theta_t, beta_t):
    nt = theta_t.shape[1]
    nb = beta_t.shape[1]
    return pl.pallas_call(
        _flatten_pair_body,
        grid=(pl.cdiv(nb, _FLAT_BLK),),
        in_specs=[
            pl.BlockSpec((1, _TH_BLK), lambda i: (0, 0)),
            pl.BlockSpec((1, _FLAT_BLK), lambda i: (0, i)),
        ],
        out_specs=[
            pl.BlockSpec((_TH_BLK,), lambda i: (0,)),
            pl.BlockSpec((_FLAT_BLK,), lambda i: (i,)),
        ],
        out_shape=[
            jax.ShapeDtypeStruct((nt,), jnp.float32),
            jax.ShapeDtypeStruct((nb,), jnp.float32),
        ],
    )(theta_t, beta_t)


@functools.partial(
    pl.kernel,
    mesh=plsc.VectorSubcoreMesh(core_axis_name="c", subcore_axis_name="s"),
    out_type=jax.ShapeDtypeStruct((_BATCH,), jnp.float32),
    scratch_types=[
        pltpu.VMEM((_B_PER_W,), jnp.int32),
        pltpu.VMEM((_B_PER_W,), jnp.int32),
        pltpu.VMEM((_B_PER_W,), jnp.float32),
        pltpu.VMEM((_B_PER_W,), jnp.float32),
        pltpu.SemaphoreType.DMA,
        pltpu.SemaphoreType.DMA,
        [pltpu.SemaphoreType.DMA] * _NCHUNK,
        [pltpu.SemaphoreType.DMA] * _NCHUNK,
        pltpu.SemaphoreType.DMA,
    ],
)
def _irt_sc(aidx_hbm, tidx_hbm, th_hbm, be_hbm, out_hbm,
            aidx_v, tidx_v, th_v, be_v, sem_ai, sem_ti, sems_th, sems_be,
            sem_o):
    wid = lax.axis_index("s") * _NC + lax.axis_index("c")
    base = wid * _B_PER_W

    cp_a = pltpu.async_copy(aidx_hbm.at[pl.ds(base, _B_PER_W)], aidx_v, sem_ai)
    cp_t = pltpu.async_copy(tidx_hbm.at[pl.ds(base, _B_PER_W)], tidx_v, sem_ti)

    cp_a.wait()
    th_cps = []
    for j in range(_NCHUNK):
        sl = pl.ds(j * _CHUNK, _CHUNK)
        th_cps.append(
            pltpu.async_copy(th_hbm.at[aidx_v.at[sl]], th_v.at[sl], sems_th[j]))
    cp_t.wait()
    be_cps = []
    for j in range(_NCHUNK):
        sl = pl.ds(j * _CHUNK, _CHUNK)
        be_cps.append(
            pltpu.async_copy(be_hbm.at[tidx_v.at[sl]], be_v.at[sl], sems_be[j]))

    out_cps = []
    for j in range(_NCHUNK):
        th_cps[j].wait()
        be_cps[j].wait()
        for i in range(_CHUNK // _LANES):
            sl = pl.ds(j * _CHUNK + i * _LANES, _LANES)
            th_v[sl] = th_v[sl] - be_v[sl]
        sl = pl.ds(j * _CHUNK, _CHUNK)
        out_cps.append(
            pltpu.async_copy(
                th_v.at[sl], out_hbm.at[pl.ds(base + j * _CHUNK, _CHUNK)],
                sem_o))
    for cp in out_cps:
        cp.wait()


def kernel(agent_idx, task_idx, theta, beta):
    agent_idx = agent_idx.astype(jnp.int32)
    task_idx = task_idx.astype(jnp.int32)
    th_flat, be_flat = _flatten_pair(theta.T, beta.T)
    return _irt_sc(agent_idx, task_idx, th_flat, be_flat)


# speedup vs baseline: 1.0658x; 1.0039x over previous
import functools

import jax
import jax.numpy as jnp
from jax import lax
from jax.experimental import pallas as pl
from jax.experimental.pallas import tpu as pltpu
from jax.experimental.pallas import tpu_sc as plsc

_BATCH = 16384

_info = plsc.get_sparse_core_info()
_NC = _info.num_cores
_NS = _info.num_subcores
_NW = _NC * _NS
_B_PER_W = _BATCH // _NW
_CHUNK = 128
_NCHUNK = _B_PER_W // _CHUNK
_LANES = 16


_FLAT_BLK = 507904


_TH_BLK = 131072


def _flatten_pair_body(th_ref, be_ref, oth_ref, obe_ref):
    i = pl.program_id(0)

    @pl.when(i == 0)
    def _():
        oth_ref[...] = th_ref[...].reshape(oth_ref.shape)

    obe_ref[...] = be_ref[...].reshape(obe_ref.shape)


def _flatten_pair(theta_t, beta_t):
    nt = theta_t.shape[1]
    nb = beta_t.shape[1]
    return pl.pallas_call(
        _flatten_pair_body,
        grid=(pl.cdiv(nb, _FLAT_BLK),),
        in_specs=[
            pl.BlockSpec((1, _TH_BLK), lambda i: (0, 0)),
            pl.BlockSpec((1, _FLAT_BLK), lambda i: (0, i)),
        ],
        out_specs=[
            pl.BlockSpec((_TH_BLK,), lambda i: (0,)),
            pl.BlockSpec((_FLAT_BLK,), lambda i: (i,)),
        ],
        out_shape=[
            jax.ShapeDtypeStruct((nt,), jnp.float32),
            jax.ShapeDtypeStruct((nb,), jnp.float32),
        ],
    )(theta_t, beta_t)


@functools.partial(
    pl.kernel,
    mesh=plsc.VectorSubcoreMesh(core_axis_name="c", subcore_axis_name="s"),
    out_type=jax.ShapeDtypeStruct((_BATCH,), jnp.float32),
    scratch_types=[
        pltpu.VMEM((_B_PER_W,), jnp.int32),
        pltpu.VMEM((_B_PER_W,), jnp.int32),
        pltpu.VMEM((_B_PER_W,), jnp.float32),
        pltpu.VMEM((_B_PER_W,), jnp.float32),
        [pltpu.SemaphoreType.DMA] * _NCHUNK,
        [pltpu.SemaphoreType.DMA] * _NCHUNK,
        [pltpu.SemaphoreType.DMA] * _NCHUNK,
        [pltpu.SemaphoreType.DMA] * _NCHUNK,
        pltpu.SemaphoreType.DMA,
    ],
)
def _irt_sc(aidx_hbm, tidx_hbm, th_hbm, be_hbm, out_hbm,
            aidx_v, tidx_v, th_v, be_v, sems_ai, sems_ti, sems_th, sems_be,
            sem_o):
    wid = lax.axis_index("s") * _NC + lax.axis_index("c")
    base = wid * _B_PER_W

    ai_cps, ti_cps = [], []
    for j in range(_NCHUNK):
        sl = pl.ds(j * _CHUNK, _CHUNK)
        hsl = pl.ds(base + j * _CHUNK, _CHUNK)
        ai_cps.append(
            pltpu.async_copy(aidx_hbm.at[hsl], aidx_v.at[sl], sems_ai[j]))
        ti_cps.append(
            pltpu.async_copy(tidx_hbm.at[hsl], tidx_v.at[sl], sems_ti[j]))
    th_cps, be_cps = [], []
    for j in range(_NCHUNK):
        sl = pl.ds(j * _CHUNK, _CHUNK)
        ai_cps[j].wait()
        th_cps.append(
            pltpu.async_copy(th_hbm.at[aidx_v.at[sl]], th_v.at[sl], sems_th[j]))
        ti_cps[j].wait()
        be_cps.append(
            pltpu.async_copy(be_hbm.at[tidx_v.at[sl]], be_v.at[sl], sems_be[j]))

    out_cps = []
    for j in range(_NCHUNK):
        th_cps[j].wait()
        be_cps[j].wait()
        for i in range(_CHUNK // _LANES):
            sl = pl.ds(j * _CHUNK + i * _LANES, _LANES)
            th_v[sl] = th_v[sl] - be_v[sl]
        sl = pl.ds(j * _CHUNK, _CHUNK)
        out_cps.append(
            pltpu.async_copy(
                th_v.at[sl], out_hbm.at[pl.ds(base + j * _CHUNK, _CHUNK)],
                sem_o))
    for cp in out_cps:
        cp.wait()


def kernel(agent_idx, task_idx, theta, beta):
    agent_idx = agent_idx.astype(jnp.int32)
    task_idx = task_idx.astype(jnp.int32)
    th_flat, be_flat = _flatten_pair(theta.T, beta.T)
    return _irt_sc(agent_idx, task_idx, th_flat, be_flat)
